# Initial kernel scaffold; baseline (speedup 1.0000x reference)
import jax
import jax.numpy as jnp
from jax.experimental import pallas as pl


def kernel(x, edge_index, batch_index, W1a, b1a, W1b, b1b, root1, bias1, W2a, b2a, W2b, b2b, root2, bias2, W3a, b3a, W3b, b3b, root3, bias3, Wfc1, bfc1, Wfc2, bfc2, Wfc3, bfc3):
    raise NotImplementedError("write your pallas kernel here")


# speedup vs baseline: 2.1156x; 2.1156x over previous
import functools

import jax
import jax.numpy as jnp
from jax import lax
from jax.experimental import pallas as pl
from jax.experimental.pallas import tpu as pltpu
from jax.experimental.pallas import tpu_sc as plsc

NC = 2
NS = 16
NW = NC * NS
CHUNK = 128


def _lrelu(v):
    return jnp.maximum(v, 0.1 * v)


def _sc_gather(table, idx_r, ncols, nch):
    e_pad = NW * nch * CHUNK
    epw = nch * CHUNK
    mesh = plsc.VectorSubcoreMesh(core_axis_name="c", subcore_axis_name="s")

    @functools.partial(
        pl.kernel,
        out_type=jax.ShapeDtypeStruct((e_pad, ncols), jnp.float32),
        mesh=mesh,
        compiler_params=pltpu.CompilerParams(use_tc_tiling_on_sc=False),
        scratch_types=[
            pltpu.VMEM((nch, CHUNK), jnp.int32),
            pltpu.VMEM((CHUNK, ncols), jnp.float32),
            pltpu.SemaphoreType.DMA,
        ],
    )
    def k(table_hbm, idx_hbm, out_hbm, idx_v, rows_v, sem):
        cid = lax.axis_index("c")
        sid = lax.axis_index("s")
        wid = cid * NS + sid
        pltpu.sync_copy(idx_hbm.at[wid], idx_v)
        base = wid * epw
        for j in range(nch):
            pltpu.async_copy(table_hbm.at[idx_v.at[j]], rows_v, sem).wait()
            pltpu.sync_copy(rows_v, out_hbm.at[pl.ds(base + j * CHUNK, CHUNK)])

    return k(table, idx_r)


def _sc_gather_pair(table, src_r, dst_r, ncols, nch):
    e_pad = NW * nch * CHUNK
    epw = nch * CHUNK
    mesh = plsc.VectorSubcoreMesh(core_axis_name="c", subcore_axis_name="s")
    ot = jax.ShapeDtypeStruct((e_pad, ncols), jnp.float32)

    @functools.partial(
        pl.kernel,
        out_type=(ot, ot),
        mesh=mesh,
        compiler_params=pltpu.CompilerParams(use_tc_tiling_on_sc=False),
        scratch_types=[
            pltpu.VMEM((nch, CHUNK), jnp.int32),
            pltpu.VMEM((nch, CHUNK), jnp.int32),
            pltpu.VMEM((CHUNK, ncols), jnp.float32),
            pltpu.VMEM((CHUNK, ncols), jnp.float32),
            pltpu.SemaphoreType.DMA,
            pltpu.SemaphoreType.DMA,
        ],
    )
    def k(table_hbm, src_hbm, dst_hbm, xs_hbm, xd_hbm,
          sidx_v, didx_v, srow_v, drow_v, sem_a, sem_b):
        cid = lax.axis_index("c")
        sid = lax.axis_index("s")
        wid = cid * NS + sid
        pltpu.sync_copy(src_hbm.at[wid], sidx_v)
        pltpu.sync_copy(dst_hbm.at[wid], didx_v)
        base = wid * epw
        for j in range(nch):
            a = pltpu.async_copy(table_hbm.at[sidx_v.at[j]], srow_v, sem_a)
            b = pltpu.async_copy(table_hbm.at[didx_v.at[j]], drow_v, sem_b)
            a.wait()
            b.wait()
            pltpu.sync_copy(srow_v, xs_hbm.at[pl.ds(base + j * CHUNK, CHUNK)])
            pltpu.sync_copy(drow_v, xd_hbm.at[pl.ds(base + j * CHUNK, CHUNK)])

    return k(table, src_r, dst_r)


def _sc_scatter_add(msg, dst_r, zeros, n_pad, ncols, nch):
    epw = nch * CHUNK
    rpt = n_pad // NS
    mesh = plsc.VectorSubcoreMesh(core_axis_name="c", subcore_axis_name="s")

    @functools.partial(
        pl.kernel,
        out_type=jax.ShapeDtypeStruct((NC, n_pad, ncols), jnp.float32),
        mesh=mesh,
        compiler_params=pltpu.CompilerParams(use_tc_tiling_on_sc=False),
        scratch_types=[
            pltpu.VMEM((nch, CHUNK), jnp.int32),
            pltpu.VMEM((CHUNK, ncols), jnp.float32),
            pltpu.VMEM_SHARED((n_pad, ncols), jnp.float32),
            pltpu.SemaphoreType.DMA,
        ],
    )
    def k(msg_hbm, idx_hbm, zero_hbm, out_hbm, idx_v, rows_v, table, sem):
        cid = lax.axis_index("c")
        sid = lax.axis_index("s")
        wid = cid * NS + sid
        pltpu.sync_copy(idx_hbm.at[wid], idx_v)
        r0 = sid * rpt
        pltpu.sync_copy(zero_hbm.at[pl.ds(r0, rpt)], table.at[pl.ds(r0, rpt)])
        plsc.subcore_barrier()
        base = wid * epw
        for j in range(nch):
            pltpu.async_copy(
                msg_hbm.at[pl.ds(base + j * CHUNK, CHUNK)], rows_v, sem
            ).wait()
            pltpu.sync_copy(rows_v, table.at[idx_v.at[j]], add=True)
        plsc.subcore_barrier()
        pltpu.sync_copy(table.at[pl.ds(r0, rpt)],
                        out_hbm.at[cid, pl.ds(r0, rpt)])

    return k(msg, dst_r, zeros)


def _msg_body(cin, cout, eb, n_real, ew_ref, hs_ref, wa_ref, ba_ref, wb_ref,
              bb_ref, out_ref):
    pid = pl.program_id(0)
    ha = _lrelu(
        jnp.dot(ew_ref[...], wa_ref[...], preferred_element_type=jnp.float32)
        + ba_ref[...])
    w = _lrelu(
        jnp.dot(ha, wb_ref[...], preferred_element_type=jnp.float32)
        + bb_ref[...])
    hs = hs_ref[...]
    acc = hs[:, 0:1] * w[:, 0:cout]
    for i in range(1, cin):
        acc = acc + hs[:, i:i + 1] * w[:, i * cout:(i + 1) * cout]
    eid = pid * eb + lax.broadcasted_iota(jnp.int32, (eb, cout), 0)
    out_ref[...] = jnp.where(eid < n_real, acc, 0.0)


def _tc_msg(ew, hs, wa, ba, wb, bb, cin, cout, eb, n_real):
    e_pad = ew.shape[0]
    grid = (e_pad // eb,)
    return pl.pallas_call(
        functools.partial(_msg_body, cin, cout, eb, n_real),
        grid=grid,
        in_specs=[
            pl.BlockSpec((eb, ew.shape[1]), lambda i: (i, 0)),
            pl.BlockSpec((eb, hs.shape[1]), lambda i: (i, 0)),
            pl.BlockSpec(wa.shape, lambda i: (0, 0)),
            pl.BlockSpec(ba.shape, lambda i: (0, 0)),
            pl.BlockSpec(wb.shape, lambda i: (0, 0)),
            pl.BlockSpec(bb.shape, lambda i: (0, 0)),
        ],
        out_specs=pl.BlockSpec((eb, cout), lambda i: (i, 0)),
        out_shape=jax.ShapeDtypeStruct((e_pad, cout), jnp.float32),
    )(ew, hs, wa, ba, wb, bb)


def _msg1_body(cin, cout, eb, n_real, xs_ref, xd_ref, wa_ref, ba_ref, wb_ref,
               bb_ref, out_ref, ew_ref):
    pid = pl.program_id(0)
    ew = xd_ref[...] - xs_ref[...]
    ew_ref[...] = ew
    ha = _lrelu(
        jnp.dot(ew, wa_ref[...], preferred_element_type=jnp.float32)
        + ba_ref[...])
    w = _lrelu(
        jnp.dot(ha, wb_ref[...], preferred_element_type=jnp.float32)
        + bb_ref[...])
    hs = xs_ref[...]
    acc = hs[:, 0:1] * w[:, 0:cout]
    for i in range(1, cin):
        acc = acc + hs[:, i:i + 1] * w[:, i * cout:(i + 1) * cout]
    eid = pid * eb + lax.broadcasted_iota(jnp.int32, (eb, cout), 0)
    out_ref[...] = jnp.where(eid < n_real, acc, 0.0)


def _tc_msg1(xs, xd, wa, ba, wb, bb, cin, cout, eb, n_real):
    e_pad = xs.shape[0]
    grid = (e_pad // eb,)
    return pl.pallas_call(
        functools.partial(_msg1_body, cin, cout, eb, n_real),
        grid=grid,
        in_specs=[
            pl.BlockSpec((eb, xs.shape[1]), lambda i: (i, 0)),
            pl.BlockSpec((eb, xd.shape[1]), lambda i: (i, 0)),
            pl.BlockSpec(wa.shape, lambda i: (0, 0)),
            pl.BlockSpec(ba.shape, lambda i: (0, 0)),
            pl.BlockSpec(wb.shape, lambda i: (0, 0)),
            pl.BlockSpec(bb.shape, lambda i: (0, 0)),
        ],
        out_specs=[
            pl.BlockSpec((eb, cout), lambda i: (i, 0)),
            pl.BlockSpec((eb, xs.shape[1]), lambda i: (i, 0)),
        ],
        out_shape=[
            jax.ShapeDtypeStruct((e_pad, cout), jnp.float32),
            jax.ShapeDtypeStruct((e_pad, xs.shape[1]), jnp.float32),
        ],
    )(xs, xd, wa, ba, wb, bb)


def _upd_body(parts_ref, hprev_ref, root_ref, bias_ref, out_ref):
    p = parts_ref[0] + parts_ref[1]
    out_ref[...] = _lrelu(
        p + jnp.dot(hprev_ref[...], root_ref[...],
                    preferred_element_type=jnp.float32) + bias_ref[...])


def _tc_update(parts, hprev, root, bias, rb):
    n_pad, c = parts.shape[1], parts.shape[2]
    grid = (n_pad // rb,)
    return pl.pallas_call(
        _upd_body,
        grid=grid,
        in_specs=[
            pl.BlockSpec((NC, rb, c), lambda i: (0, i, 0)),
            pl.BlockSpec((rb, hprev.shape[1]), lambda i: (i, 0)),
            pl.BlockSpec(root.shape, lambda i: (0, 0)),
            pl.BlockSpec(bias.shape, lambda i: (0, 0)),
        ],
        out_specs=pl.BlockSpec((rb, c), lambda i: (i, 0)),
        out_shape=jax.ShapeDtypeStruct((n_pad, c), jnp.float32),
    )(parts, hprev, root, bias)


def _ro_body(nblocks, nseg, rb, h_ref, bi_ref, w1_ref, b1_ref, w2_ref, b2_ref,
             w3_ref, b3_ref, out_ref, acc_ref):
    i = pl.program_id(0)

    @pl.when(i == 0)
    def _():
        acc_ref[...] = jnp.zeros_like(acc_ref)

    bi = bi_ref[0, 0, :]
    oh = (bi[None, :] == lax.broadcasted_iota(jnp.int32, (nseg, rb), 0)
          ).astype(jnp.float32)
    acc_ref[...] += jnp.dot(oh, h_ref[...],
                            preferred_element_type=jnp.float32)

    @pl.when(i == nblocks - 1)
    def _():
        g = acc_ref[...]
        a = _lrelu(jnp.dot(g, w1_ref[...],
                           preferred_element_type=jnp.float32) + b1_ref[...])
        a = _lrelu(jnp.dot(a, w2_ref[...],
                           preferred_element_type=jnp.float32) + b2_ref[...])
        out_ref[...] = jnp.dot(a, w3_ref[...],
                               preferred_element_type=jnp.float32) + b3_ref[...]


def _tc_readout(h, bi3, wfc1, bfc1, wfc2, bfc2, wfc3, bfc3, rb, nseg):
    n_pad, c = h.shape
    nblocks = n_pad // rb
    return pl.pallas_call(
        functools.partial(_ro_body, nblocks, nseg, rb),
        grid=(nblocks,),
        in_specs=[
            pl.BlockSpec((rb, c), lambda i: (i, 0)),
            pl.BlockSpec((1, 1, rb), lambda i: (i, 0, 0)),
            pl.BlockSpec(wfc1.shape, lambda i: (0, 0)),
            pl.BlockSpec(bfc1.shape, lambda i: (0, 0)),
            pl.BlockSpec(wfc2.shape, lambda i: (0, 0)),
            pl.BlockSpec(bfc2.shape, lambda i: (0, 0)),
            pl.BlockSpec(wfc3.shape, lambda i: (0, 0)),
            pl.BlockSpec(bfc3.shape, lambda i: (0, 0)),
        ],
        out_specs=pl.BlockSpec((nseg, 1), lambda i: (0, 0)),
        out_shape=jax.ShapeDtypeStruct((nseg, 1), jnp.float32),
        scratch_shapes=[pltpu.VMEM((nseg, c), jnp.float32)],
    )(h, bi3, wfc1, bfc1, wfc2, bfc2, wfc3, bfc3)


def kernel(x, edge_index, batch_index, W1a, b1a, W1b, b1b, root1, bias1,
           W2a, b2a, W2b, b2b, root2, bias2, W3a, b3a, W3b, b3b, root3,
           bias3, Wfc1, bfc1, Wfc2, bfc2, Wfc3, bfc3):
    f32 = jnp.float32
    n = x.shape[0]
    e = edge_index.shape[1]
    nseg = 32

    nch = -(-e // (NW * CHUNK))
    e_pad = NW * nch * CHUNK
    n_pad = -(-n // 2048) * 2048
    rb = 512

    src = edge_index[0]
    dst = edge_index[1]
    pad_idx = (jnp.arange(e_pad - e, dtype=jnp.int32) * 97) % n
    src_r = jnp.concatenate([src, pad_idx]).reshape(NW, nch, CHUNK)
    dst_r = jnp.concatenate([dst, pad_idx]).reshape(NW, nch, CHUNK)
    xp = jnp.zeros((n_pad, 8), f32).at[:n, :4].set(x)
    bi3 = (jnp.full((n_pad,), nseg, jnp.int32).at[:n].set(batch_index)
           .reshape(n_pad // rb, 1, rb))

    w1a = jnp.zeros((8, 64), f32).at[1:4].set(W1a)
    w2a = jnp.zeros((8, 64), f32).at[1:4].set(W2a)
    w3a = jnp.zeros((8, 64), f32).at[1:4].set(W3a)
    w1b = jnp.zeros((64, 64), f32).at[:, :32].set(W1b)
    b1b_p = jnp.zeros((1, 64), f32).at[0, :32].set(b1b)
    r1 = jnp.zeros((8, 8), f32).at[:4].set(root1)

    b1a_2 = b1a.reshape(1, -1)
    b2a_2 = b2a.reshape(1, -1)
    b3a_2 = b3a.reshape(1, -1)
    b2b_2 = b2b.reshape(1, -1)
    b3b_2 = b3b.reshape(1, -1)
    bias1_2 = bias1.reshape(1, -1)
    bias2_2 = bias2.reshape(1, -1)
    bias3_2 = bias3.reshape(1, -1)
    bfc1_2 = bfc1.reshape(1, -1)
    bfc2_2 = bfc2.reshape(1, -1)
    bfc3_2 = bfc3.reshape(1, -1)

    z8 = jnp.zeros((n_pad, 8), f32)
    z64 = jnp.zeros((n_pad, 64), f32)
    z128 = jnp.zeros((n_pad, 128), f32)

    xs, xd = _sc_gather_pair(xp, src_r, dst_r, 8, nch)
    msg1, ew8 = _tc_msg1(xs, xd, w1a, b1a_2, w1b, b1b_p, 8, 8, 2048, e)
    parts1 = _sc_scatter_add(msg1, dst_r, z8, n_pad, 8, nch)
    h1 = _tc_update(parts1, xp, r1, bias1_2, rb)

    hs2 = _sc_gather(h1, src_r, 8, nch)
    msg2 = _tc_msg(ew8, hs2, w2a, b2a_2, W2b, b2b_2, 8, 64, 1024, e)
    parts2 = _sc_scatter_add(msg2, dst_r, z64, n_pad, 64, nch)
    h2 = _tc_update(parts2, h1, root2, bias2_2, rb)

    hs3 = _sc_gather(h2, src_r, 64, nch)
    msg3 = _tc_msg(ew8, hs3, w3a, b3a_2, W3b, b3b_2, 64, 128, 256, e)
    parts3 = _sc_scatter_add(msg3, dst_r, z128, n_pad, 128, nch)
    h3 = _tc_update(parts3, h2, root3, bias3_2, rb)

    return _tc_readout(h3, bi3, Wfc1, bfc1_2, Wfc2, bfc2_2, Wfc3, bfc3_2,
                       rb, nseg)
